# baseline (speedup 1.0000x reference)
import functools

import jax
import jax.numpy as jnp
from jax import lax
from jax.experimental import pallas as pl
from jax.experimental.pallas import tpu as pltpu
from jax.experimental.pallas import tpu_sc as plsc

_FIELDS = 26
_VOCAB = 100000
_EMB = 32
_LANES = 16


def _body(rows_per_w, nc, x_hbm, tab_hbm, out_hbm, xv, idx_all, rows_v, sems):
    wid = lax.axis_index("s") * nc + lax.axis_index("c")
    b0 = wid * rows_per_w
    lane = lax.iota(jnp.int32, _LANES)

    pltpu.sync_copy(x_hbm.at[pl.ds(b0, rows_per_w)], xv)

    for f in range(_FIELDS):
        col = jnp.full((_LANES,), f, jnp.int32)

        def extract(j, _):
            rows = j * _LANES + lane
            idx_all[f, pl.ds(j * _LANES, _LANES)] = plsc.load_gather(
                xv, [rows, col])
            return 0

        lax.fori_loop(0, rows_per_w // _LANES, extract, 0, unroll=4)

    def gather(f, slot):
        return pltpu.async_copy(tab_hbm.at[f].at[idx_all.at[f]],
                                rows_v.at[slot], sems.at[slot])

    def writeback(f, slot):
        pltpu.sync_copy(rows_v.at[slot],
                        out_hbm.at[pl.ds(b0, rows_per_w),
                                   pl.ds(f * _EMB, _EMB)])

    inflight = {0: gather(0, 0)}
    for f in range(_FIELDS):
        nxt = f + 1
        if nxt < _FIELDS:
            inflight[nxt] = gather(nxt, nxt % 2)
        inflight.pop(f).wait()
        writeback(f, f % 2)


def kernel(x, tables):
    batch = x.shape[0]

    info = plsc.get_sparse_core_info()
    nc, ns = info.num_cores, info.num_subcores
    nw = nc * ns
    rows_per_w = batch // nw

    mesh = plsc.VectorSubcoreMesh(core_axis_name="c", subcore_axis_name="s")
    run = pl.kernel(
        functools.partial(_body, rows_per_w, nc),
        out_type=jax.ShapeDtypeStruct((batch, _FIELDS * _EMB), jnp.float32),
        mesh=mesh,
        compiler_params=pltpu.CompilerParams(use_tc_tiling_on_sc=False,
                                             needs_layout_passes=False),
        scratch_types=[
            pltpu.VMEM((512, _FIELDS), jnp.int32),
            pltpu.VMEM((_FIELDS, 512), jnp.int32),
            pltpu.VMEM((2, 512, _EMB), jnp.float32),
            pltpu.SemaphoreType.DMA((2,)),
        ],
    )
    return run(x, tables)


# speedup vs baseline: 1.4288x; 1.4288x over previous
import functools

import jax
import jax.numpy as jnp
from jax import lax
from jax.experimental import pallas as pl
from jax.experimental.pallas import tpu as pltpu
from jax.experimental.pallas import tpu_sc as plsc

_FIELDS = 26
_VOCAB = 100000
_EMB = 32
_LANES = 16
_B = 16384
_COLS = 896
_MESH = dict(core_axis_name="c", subcore_axis_name="s")


def _xcols_body(nc, x_hbm, xcols_hbm, xv, idx_all):
    wid = lax.axis_index("s") * nc + lax.axis_index("c")
    b0 = wid * 512
    lane = lax.iota(jnp.int32, _LANES)

    pltpu.sync_copy(x_hbm.at[pl.ds(b0, 512)], xv)

    for f in range(_FIELDS):
        col = jnp.full((_LANES,), f, jnp.int32)

        def extract(j, _):
            rows = j * _LANES + lane
            idx_all[f, pl.ds(j * _LANES, _LANES)] = plsc.load_gather(
                xv, [rows, col])
            return 0

        lax.fori_loop(0, 512 // _LANES, extract, 0, unroll=4)
        pltpu.sync_copy(idx_all.at[f],
                        xcols_hbm.at[pl.ds(f * _B + b0, 512)])


def _gather_body(nc, tab_hbm, xcols_hbm, outT_hbm, row_v, xi_v, ob_v, sem):
    wid = lax.axis_index("s") * nc + lax.axis_index("c")
    n_items = _FIELDS * _EMB // 32

    for m in range(n_items):
        g = wid + m * 32
        f = g // _EMB
        cp = pltpu.async_copy(tab_hbm.at[f, g % _EMB, :], row_v, sem)
        for c in range(2):
            pltpu.sync_copy(
                xcols_hbm.at[pl.ds(f * _B + c * 8192, 8192)], xi_v)
            if c == 0:
                cp.wait()

            def gat(j, _):
                sl = pl.ds(j * _LANES, _LANES)
                ob_v[sl] = plsc.load_gather(row_v, [xi_v[sl]])
                return 0

            lax.fori_loop(0, 8192 // _LANES, gat, 0, unroll=4)
            pltpu.sync_copy(ob_v, outT_hbm.at[g, pl.ds(c * 8192, 8192)])


def _transpose_body(nc, outT_hbm, out_hbm, in128, out128):
    wid = lax.axis_index("s") * nc + lax.axis_index("c")
    lane = lax.iota(jnp.int32, _LANES)
    n_items = (_COLS // 128) * (_B // 128) // 32

    for m in range(n_items):
        g = wid + m * 32
        fq = g // (_B // 128)
        bb = g % (_B // 128)
        pltpu.sync_copy(
            outT_hbm.at[pl.ds(fq * 128, 128), pl.ds(bb * 128, 128)], in128)

        def tp(j, _):
            b = j // 8
            cc = (j % 8) * _LANES + lane
            vals = plsc.load_gather(in128, [cc, jnp.full((_LANES,), b,
                                                         jnp.int32)])
            plsc.store_scatter(out128,
                               [jnp.full((_LANES,), b, jnp.int32), cc], vals)
            return 0

        lax.fori_loop(0, 1024, tp, 0, unroll=4)
        pltpu.sync_copy(
            out128, out_hbm.at[pl.ds(bb * 128, 128), pl.ds(fq * 128, 128)])


def kernel(x, tables):
    tab_t = jnp.swapaxes(tables, 1, 2)

    info = plsc.get_sparse_core_info()
    nc = info.num_cores
    mesh = plsc.VectorSubcoreMesh(**_MESH)

    xcols = pl.kernel(
        functools.partial(_xcols_body, nc),
        out_type=jax.ShapeDtypeStruct((_FIELDS * _B,), jnp.int32),
        mesh=mesh,
        compiler_params=pltpu.CompilerParams(use_tc_tiling_on_sc=False,
                                             needs_layout_passes=False),
        scratch_types=[
            pltpu.VMEM((512, _FIELDS), jnp.int32),
            pltpu.VMEM((_FIELDS, 512), jnp.int32),
        ],
    )(x)

    outT = pl.kernel(
        functools.partial(_gather_body, nc),
        out_type=jax.ShapeDtypeStruct((_COLS, _B), jnp.float32),
        mesh=mesh,
        compiler_params=pltpu.CompilerParams(use_tc_tiling_on_sc=True,
                                             needs_layout_passes=False),
        scratch_types=[
            pltpu.VMEM((_VOCAB,), jnp.float32),
            pltpu.VMEM((8192,), jnp.int32),
            pltpu.VMEM((8192,), jnp.float32),
            pltpu.SemaphoreType.DMA,
        ],
    )(tab_t, xcols)

    out = pl.kernel(
        functools.partial(_transpose_body, nc),
        out_type=jax.ShapeDtypeStruct((_B, _COLS), jnp.float32),
        mesh=mesh,
        compiler_params=pltpu.CompilerParams(use_tc_tiling_on_sc=True,
                                             needs_layout_passes=False),
        scratch_types=[
            pltpu.VMEM((128, 128), jnp.float32),
            pltpu.VMEM((128, 128), jnp.float32),
        ],
    )(outT)

    return out[:, :_FIELDS * _EMB]


# speedup vs baseline: 1.5129x; 1.0588x over previous
import functools

import jax
import jax.numpy as jnp
from jax import lax
from jax.experimental import pallas as pl
from jax.experimental.pallas import tpu as pltpu
from jax.experimental.pallas import tpu_sc as plsc

_FIELDS = 26
_VOCAB = 100000
_EMB = 32
_LANES = 16
_B = 16384
_COLS = 896
_MESH = dict(core_axis_name="c", subcore_axis_name="s")


def _xcols_body(nc, x_hbm, xcols_hbm, xv, idx_all):
    wid = lax.axis_index("s") * nc + lax.axis_index("c")
    b0 = wid * 512
    lane = lax.iota(jnp.int32, _LANES)

    pltpu.sync_copy(x_hbm.at[pl.ds(b0, 512)], xv)

    for f in range(_FIELDS):
        col = jnp.full((_LANES,), f, jnp.int32)

        def extract(j, _):
            rows = j * _LANES + lane
            idx_all[f, pl.ds(j * _LANES, _LANES)] = plsc.load_gather(
                xv, [rows, col])
            return 0

        lax.fori_loop(0, 512 // _LANES, extract, 0, unroll=4)
        pltpu.sync_copy(idx_all.at[f],
                        xcols_hbm.at[pl.ds(f * _B + b0, 512)])


def _gather_body(nc, tab_hbm, xcols_hbm, outT_hbm, row_v, xi_v, ob_v,
                 rsem, osems):
    wid = lax.axis_index("s") * nc + lax.axis_index("c")
    n_items = _FIELDS * _EMB // 32

    obflight = {}
    for m in range(n_items):
        g = wid + m * 32
        f = g // _EMB
        cp = pltpu.async_copy(tab_hbm.at[f, g % _EMB, :], row_v, rsem)
        for c in range(2):
            pltpu.sync_copy(
                xcols_hbm.at[pl.ds(f * _B + c * 8192, 8192)], xi_v)
            if c == 0:
                cp.wait()
            if (m - 1, c) in obflight:
                obflight.pop((m - 1, c)).wait()

            def gat(j, _):
                sl = pl.ds(j * _LANES, _LANES)
                ob_v[c, sl] = plsc.load_gather(row_v, [xi_v[sl]])
                return 0

            lax.fori_loop(0, 8192 // _LANES, gat, 0, unroll=4)
            obflight[(m, c)] = pltpu.async_copy(
                ob_v.at[c], outT_hbm.at[g, pl.ds(c * 8192, 8192)],
                osems.at[c])
    for h in obflight.values():
        h.wait()


def _transpose_body(nc, outT_hbm, out_hbm, in_v, out_v, isems, osems):
    wid = lax.axis_index("s") * nc + lax.axis_index("c")
    lane = lax.iota(jnp.int32, _LANES)
    ccs = [j2 * _LANES + lane for j2 in range(8)]
    n_items = (_COLS // 128) * (_B // 128) // 32

    def coords(m):
        g = wid + m * 32
        return g // (_B // 128), g % (_B // 128)

    def load(m, slot):
        fq, bb = coords(m)
        return pltpu.async_copy(
            outT_hbm.at[pl.ds(fq * 128, 128), pl.ds(bb * 128, 128)],
            in_v.at[slot], isems.at[slot])

    inflight = {0: load(0, 0)}
    outflight = {}
    for m in range(n_items):
        slot = m % 2
        if m + 1 < n_items:
            inflight[m + 1] = load(m + 1, (m + 1) % 2)
        inflight.pop(m).wait()
        if m - 2 in outflight:
            outflight.pop(m - 2).wait()

        def tp_b(b, _):
            bs = jnp.full((_LANES,), b, jnp.int32)
            for j2 in range(8):
                vals = plsc.load_gather(in_v.at[slot], [ccs[j2], bs])
                plsc.store_scatter(out_v.at[slot], [bs, ccs[j2]], vals)
            return 0

        lax.fori_loop(0, 128, tp_b, 0, unroll=2)
        fq, bb = coords(m)
        outflight[m] = pltpu.async_copy(
            out_v.at[slot],
            out_hbm.at[pl.ds(bb * 128, 128), pl.ds(fq * 128, 128)],
            osems.at[slot])
    for h in outflight.values():
        h.wait()


def kernel(x, tables):
    tab_t = jnp.swapaxes(tables, 1, 2)

    info = plsc.get_sparse_core_info()
    nc = info.num_cores
    mesh = plsc.VectorSubcoreMesh(**_MESH)

    xcols = pl.kernel(
        functools.partial(_xcols_body, nc),
        out_type=jax.ShapeDtypeStruct((_FIELDS * _B,), jnp.int32),
        mesh=mesh,
        compiler_params=pltpu.CompilerParams(use_tc_tiling_on_sc=False,
                                             needs_layout_passes=False),
        scratch_types=[
            pltpu.VMEM((512, _FIELDS), jnp.int32),
            pltpu.VMEM((_FIELDS, 512), jnp.int32),
        ],
    )(x)

    outT = pl.kernel(
        functools.partial(_gather_body, nc),
        out_type=jax.ShapeDtypeStruct((_COLS, _B), jnp.float32),
        mesh=mesh,
        compiler_params=pltpu.CompilerParams(use_tc_tiling_on_sc=True,
                                             needs_layout_passes=False),
        scratch_types=[
            pltpu.VMEM((_VOCAB,), jnp.float32),
            pltpu.VMEM((8192,), jnp.int32),
            pltpu.VMEM((2, 8192), jnp.float32),
            pltpu.SemaphoreType.DMA,
            pltpu.SemaphoreType.DMA((2,)),
        ],
    )(tab_t, xcols)

    out = pl.kernel(
        functools.partial(_transpose_body, nc),
        out_type=jax.ShapeDtypeStruct((_B, _COLS), jnp.float32),
        mesh=mesh,
        compiler_params=pltpu.CompilerParams(use_tc_tiling_on_sc=True,
                                             needs_layout_passes=False),
        scratch_types=[
            pltpu.VMEM((2, 128, 128), jnp.float32),
            pltpu.VMEM((2, 128, 128), jnp.float32),
            pltpu.SemaphoreType.DMA((2,)),
            pltpu.SemaphoreType.DMA((2,)),
        ],
    )(outT)

    return out[:, :_FIELDS * _EMB]


# speedup vs baseline: 1.6315x; 1.0784x over previous
import functools

import jax
import jax.numpy as jnp
from jax import lax
from jax.experimental import pallas as pl
from jax.experimental.pallas import tpu as pltpu
from jax.experimental.pallas import tpu_sc as plsc

_FIELDS = 26
_VOCAB = 100000
_EMB = 32
_LANES = 16
_B = 16384
_COLS = 896
_MESH = dict(core_axis_name="c", subcore_axis_name="s")


def _xcols_body(nc, x_hbm, xcols_hbm, xv, idx_all):
    wid = lax.axis_index("s") * nc + lax.axis_index("c")
    b0 = wid * 512
    lane = lax.iota(jnp.int32, _LANES)

    pltpu.sync_copy(x_hbm.at[pl.ds(b0, 512)], xv)

    for f in range(_FIELDS):
        col = jnp.full((_LANES,), f, jnp.int32)

        def extract(j, _):
            rows = j * _LANES + lane
            idx_all[f, pl.ds(j * _LANES, _LANES)] = plsc.load_gather(
                xv, [rows, col])
            return 0

        lax.fori_loop(0, 512 // _LANES, extract, 0, unroll=4)
        pltpu.sync_copy(idx_all.at[f],
                        xcols_hbm.at[pl.ds(f * _B + b0, 512)])


def _gather_body(nc, tab_hbm, xcols_hbm, outT_hbm, row_v, xi_v, ob_v,
                 rsem, osems):
    wid = lax.axis_index("s") * nc + lax.axis_index("c")
    n_items = _FIELDS * _EMB // 32

    obflight = {}
    for m in range(n_items):
        g = wid + m * 32
        f = g // _EMB
        cp = pltpu.async_copy(tab_hbm.at[f, g % _EMB, :], row_v, rsem)
        for c in range(2):
            pltpu.sync_copy(
                xcols_hbm.at[pl.ds(f * _B + c * 8192, 8192)], xi_v)
            if c == 0:
                cp.wait()
            if (m - 1, c) in obflight:
                obflight.pop((m - 1, c)).wait()

            def gat(j, _):
                sl = pl.ds(j * _LANES, _LANES)
                ob_v[c, sl] = plsc.load_gather(row_v, [xi_v[sl]])
                return 0

            lax.fori_loop(0, 8192 // _LANES, gat, 0, unroll=4)
            obflight[(m, c)] = pltpu.async_copy(
                ob_v.at[c], outT_hbm.at[g, pl.ds(c * 8192, 8192)],
                osems.at[c])
    for h in obflight.values():
        h.wait()


def _transpose_body(nc, outT_hbm, out_hbm, in_v, out_v, isems, osems):
    wid = lax.axis_index("s") * nc + lax.axis_index("c")
    lane = lax.iota(jnp.int32, _LANES)
    ccs = [j2 * _LANES + lane for j2 in range(8)]
    n_items = (_COLS // 128) * (_B // 128) // 32

    def coords(m):
        g = wid + m * 32
        return g // (_B // 128), g % (_B // 128)

    def load(m, slot):
        fq, bb = coords(m)
        return pltpu.async_copy(
            outT_hbm.at[pl.ds(fq * 128, 128), pl.ds(bb * 128, 128)],
            in_v.at[slot], isems.at[slot])

    inflight = {0: load(0, 0)}
    outflight = {}
    for m in range(n_items):
        slot = m % 2
        if m + 1 < n_items:
            inflight[m + 1] = load(m + 1, (m + 1) % 2)
        inflight.pop(m).wait()
        if m - 2 in outflight:
            outflight.pop(m - 2).wait()

        def tp_cc(cc, _):
            ccs_ = jnp.full((_LANES,), cc, jnp.int32)
            for j2 in range(8):
                vals = plsc.load_gather(in_v.at[slot], [ccs_, ccs[j2]])
                plsc.store_scatter(out_v.at[slot], [ccs[j2], ccs_], vals)
            return 0

        lax.fori_loop(0, 128, tp_cc, 0, unroll=2)
        fq, bb = coords(m)
        outflight[m] = pltpu.async_copy(
            out_v.at[slot, :, pl.ds(0, 128)],
            out_hbm.at[pl.ds(bb * 128, 128), pl.ds(fq * 128, 128)],
            osems.at[slot])
    for h in outflight.values():
        h.wait()


def kernel(x, tables):
    tab_t = jnp.swapaxes(tables, 1, 2)

    info = plsc.get_sparse_core_info()
    nc = info.num_cores
    mesh = plsc.VectorSubcoreMesh(**_MESH)

    xcols = pl.kernel(
        functools.partial(_xcols_body, nc),
        out_type=jax.ShapeDtypeStruct((_FIELDS * _B,), jnp.int32),
        mesh=mesh,
        compiler_params=pltpu.CompilerParams(use_tc_tiling_on_sc=False,
                                             needs_layout_passes=False),
        scratch_types=[
            pltpu.VMEM((512, _FIELDS), jnp.int32),
            pltpu.VMEM((_FIELDS, 512), jnp.int32),
        ],
    )(x)

    outT = pl.kernel(
        functools.partial(_gather_body, nc),
        out_type=jax.ShapeDtypeStruct((_COLS, _B), jnp.float32),
        mesh=mesh,
        compiler_params=pltpu.CompilerParams(use_tc_tiling_on_sc=True,
                                             needs_layout_passes=False),
        scratch_types=[
            pltpu.VMEM((_VOCAB,), jnp.float32),
            pltpu.VMEM((8192,), jnp.int32),
            pltpu.VMEM((2, 8192), jnp.float32),
            pltpu.SemaphoreType.DMA,
            pltpu.SemaphoreType.DMA((2,)),
        ],
    )(tab_t, xcols)

    out = pl.kernel(
        functools.partial(_transpose_body, nc),
        out_type=jax.ShapeDtypeStruct((_B, _COLS), jnp.float32),
        mesh=mesh,
        compiler_params=pltpu.CompilerParams(use_tc_tiling_on_sc=True,
                                             needs_layout_passes=False),
        scratch_types=[
            pltpu.VMEM((2, 128, 128), jnp.float32),
            pltpu.VMEM((2, 128, 129), jnp.float32),
            pltpu.SemaphoreType.DMA((2,)),
            pltpu.SemaphoreType.DMA((2,)),
        ],
    )(outT)

    return out[:, :_FIELDS * _EMB]


# speedup vs baseline: 1.7535x; 1.0748x over previous
import functools

import jax
import jax.numpy as jnp
from jax import lax
from jax.experimental import pallas as pl
from jax.experimental.pallas import tpu as pltpu
from jax.experimental.pallas import tpu_sc as plsc

_FIELDS = 26
_VOCAB = 100000
_EMB = 32
_LANES = 16
_B = 16384
_COLS = 896
_MESH = dict(core_axis_name="c", subcore_axis_name="s")


def _xcols_body(nc, x_hbm, xcols_hbm, xv, idx_all):
    wid = lax.axis_index("s") * nc + lax.axis_index("c")
    b0 = wid * 512
    lane = lax.iota(jnp.int32, _LANES)

    pltpu.sync_copy(x_hbm.at[pl.ds(b0, 512)], xv)

    for f in range(_FIELDS):
        col = jnp.full((_LANES,), f, jnp.int32)

        def extract(j, _):
            rows = j * _LANES + lane
            idx_all[f, pl.ds(j * _LANES, _LANES)] = plsc.load_gather(
                xv, [rows, col])
            return 0

        lax.fori_loop(0, 512 // _LANES, extract, 0, unroll=4)
        pltpu.sync_copy(idx_all.at[f],
                        xcols_hbm.at[pl.ds(f * _B + b0, 512)])


def _gather_body(nc, tab_hbm, xcols_hbm, outT_hbm, row_v, xi_v, ob_v,
                 rsem, osems, xsems):
    wid = lax.axis_index("s") * nc + lax.axis_index("c")
    n_items = _FIELDS * _EMB // 32

    nch = 4
    csz = _B // nch

    def xi_load(f, c):
        return pltpu.async_copy(
            xcols_hbm.at[pl.ds(f * _B + c * csz, csz)],
            xi_v.at[c % 2], xsems.at[c % 2])

    obflight = {}
    xiflight = None
    for m in range(n_items):
        g = wid + m * 32
        f = g // _EMB
        cp = pltpu.async_copy(tab_hbm.at[f, g % _EMB, :], row_v, rsem)
        if xiflight is None:
            xiflight = xi_load(f, 0)
        for c in range(nch):
            if c + 1 < nch:
                nxt = xi_load(f, c + 1)
            elif m + 1 < n_items:
                nxt = xi_load((g + 32) // _EMB, 0)
            else:
                nxt = None
            xiflight.wait()
            xiflight = nxt
            if c == 0:
                cp.wait()
            cslot = c % 2
            if (m, c - 2) in obflight:
                obflight.pop((m, c - 2)).wait()
            if (m - 1, c + nch - 2) in obflight:
                obflight.pop((m - 1, c + nch - 2)).wait()

            def gat(j, _):
                sl = pl.ds(j * _LANES, _LANES)
                ob_v[cslot, sl] = plsc.load_gather(row_v,
                                                   [xi_v[cslot, sl]])
                return 0

            lax.fori_loop(0, csz // _LANES, gat, 0, unroll=4)
            obflight[(m, c)] = pltpu.async_copy(
                ob_v.at[cslot], outT_hbm.at[g, pl.ds(c * csz, csz)],
                osems.at[cslot])
    for h in obflight.values():
        h.wait()


def _transpose_body(nc, outT_hbm, out_hbm, in_v, out_v, isems, osems):
    wid = lax.axis_index("s") * nc + lax.axis_index("c")
    lane = lax.iota(jnp.int32, _LANES)
    ccs = [j2 * _LANES + lane for j2 in range(8)]
    n_items = (_COLS // 128) * (_B // 128) // 32

    def coords(m):
        g = wid + m * 32
        return g // (_B // 128), g % (_B // 128)

    def load(m, slot):
        fq, bb = coords(m)
        return pltpu.async_copy(
            outT_hbm.at[pl.ds(fq * 128, 128), pl.ds(bb * 128, 128)],
            in_v.at[slot], isems.at[slot])

    inflight = {0: load(0, 0)}
    outflight = {}
    for m in range(n_items):
        slot = m % 2
        if m + 1 < n_items:
            inflight[m + 1] = load(m + 1, (m + 1) % 2)
        inflight.pop(m).wait()
        if m - 2 in outflight:
            outflight.pop(m - 2).wait()

        def tp_cc(cc, _):
            ccs_ = jnp.full((_LANES,), cc, jnp.int32)
            for j2 in range(8):
                vals = plsc.load_gather(in_v.at[slot], [ccs_, ccs[j2]])
                plsc.store_scatter(out_v.at[slot], [ccs[j2], ccs_], vals)
            return 0

        lax.fori_loop(0, 128, tp_cc, 0, unroll=2)
        fq, bb = coords(m)
        outflight[m] = pltpu.async_copy(
            out_v.at[slot, :, pl.ds(0, 128)],
            out_hbm.at[pl.ds(bb * 128, 128), pl.ds(fq * 128, 128)],
            osems.at[slot])
    for h in outflight.values():
        h.wait()


def kernel(x, tables):
    tab_t = jnp.swapaxes(tables, 1, 2)

    info = plsc.get_sparse_core_info()
    nc = info.num_cores
    mesh = plsc.VectorSubcoreMesh(**_MESH)

    xcols = pl.kernel(
        functools.partial(_xcols_body, nc),
        out_type=jax.ShapeDtypeStruct((_FIELDS * _B,), jnp.int32),
        mesh=mesh,
        compiler_params=pltpu.CompilerParams(use_tc_tiling_on_sc=False,
                                             needs_layout_passes=False),
        scratch_types=[
            pltpu.VMEM((512, _FIELDS), jnp.int32),
            pltpu.VMEM((_FIELDS, 512), jnp.int32),
        ],
    )(x)

    outT = pl.kernel(
        functools.partial(_gather_body, nc),
        out_type=jax.ShapeDtypeStruct((_COLS, _B), jnp.float32),
        mesh=mesh,
        compiler_params=pltpu.CompilerParams(use_tc_tiling_on_sc=True,
                                             needs_layout_passes=False),
        scratch_types=[
            pltpu.VMEM((_VOCAB,), jnp.float32),
            pltpu.VMEM((2, 4096), jnp.int32),
            pltpu.VMEM((2, 4096), jnp.float32),
            pltpu.SemaphoreType.DMA,
            pltpu.SemaphoreType.DMA((2,)),
            pltpu.SemaphoreType.DMA((2,)),
        ],
    )(tab_t, xcols)

    out = pl.kernel(
        functools.partial(_transpose_body, nc),
        out_type=jax.ShapeDtypeStruct((_B, _COLS), jnp.float32),
        mesh=mesh,
        compiler_params=pltpu.CompilerParams(use_tc_tiling_on_sc=True,
                                             needs_layout_passes=False),
        scratch_types=[
            pltpu.VMEM((2, 128, 128), jnp.float32),
            pltpu.VMEM((2, 128, 129), jnp.float32),
            pltpu.SemaphoreType.DMA((2,)),
            pltpu.SemaphoreType.DMA((2,)),
        ],
    )(outT)

    return out[:, :_FIELDS * _EMB]
